# Initial kernel scaffold; baseline (speedup 1.0000x reference)
import jax
import jax.numpy as jnp
from jax.experimental import pallas as pl


def kernel(x, reference_point, c, Wg, bg, Wfc, bfc, Wproj, bproj):
    raise NotImplementedError("write your pallas kernel here")


# speedup vs baseline: 2.7143x; 2.7143x over previous
import functools

import jax
import jax.numpy as jnp
from jax.experimental import pallas as pl

N_EXPERTS = 32
N_EMBD = 256
HIDDEN = 4 * N_EMBD
TOPK = 7

BTG = 256
BT = 256
T = 2048

_INV_SQRT2 = 0.7071067811865476


def _gate_kernel(x_ref, wgt_ref, bgp_ref, rw_ref):
    xb = x_ref[...]
    logits = jnp.dot(xb, wgt_ref[...], preferred_element_type=jnp.float32)
    logits = logits + bgp_ref[...]
    m = jnp.max(logits, axis=1, keepdims=True)
    ex = jnp.exp(logits - m)
    z = jnp.sum(ex, axis=1, keepdims=True)
    p = ex / z
    lane = jax.lax.broadcasted_iota(jnp.int32, (BTG, 128), 1)
    valid = lane < (N_EXPERTS - 1)
    p = jnp.where(valid, p, -1.0)
    pcur = p
    selmask = jnp.zeros((BTG, 128), dtype=jnp.bool_)
    for _ in range(TOPK):
        mj = jnp.max(pcur, axis=1, keepdims=True)
        ismax = pcur == mj
        selidx = jnp.min(jnp.where(ismax, lane, 127), axis=1, keepdims=True)
        mask_j = lane == selidx
        selmask = jnp.logical_or(selmask, mask_j)
        pcur = jnp.where(mask_j, -1.0, pcur)
    psel = jnp.where(selmask, p, 0.0)
    s = jnp.sum(psel, axis=1, keepdims=True)
    scaled = psel * ((TOPK / (TOPK + 1.0)) / s)
    col0 = jnp.full((BTG, 1), 1.0 / (TOPK + 1.0), dtype=jnp.float32)
    rw_ref[...] = jnp.concatenate([col0, scaled[:, : N_EXPERTS - 1]], axis=1)


def _expmap(rb, y, cv):
    xn2 = jnp.sum(rb * rb, axis=-1, keepdims=True)
    sf = 2.0 / (1.0 + cv * xn2)
    vn2 = jnp.sum(y * y, axis=-1, keepdims=True)
    vn = jnp.sqrt(vn2)
    arg = jnp.sqrt(cv * sf * vn2 / 2.0)
    second = (1.0 / jnp.sqrt(cv)) * jnp.tanh(arg) * y / vn
    ip = jnp.sum(rb * second, axis=-1, keepdims=True)
    sn2 = jnp.sum(second * second, axis=-1, keepdims=True)
    num = (1.0 + 2.0 * cv * ip + cv * sn2) * rb + (1.0 - cv * xn2) * second
    den = 1.0 + 2.0 * cv * ip + cv * cv * xn2 * sn2
    return num / den


def _dense_kernel(x_ref, ref_ref, rw_ref, c_ref, wfc_ref, bfc_ref,
                  wproj_ref, bproj_ref, out_ref):
    e = pl.program_id(0)
    t = pl.program_id(1)
    sl = pl.ds(t * BT, BT)
    xb = x_ref[sl, :]
    h = jax.lax.dot_general(xb, wfc_ref[0], (((1,), (1,)), ((), ())),
                            preferred_element_type=jnp.float32)
    h = h + bfc_ref[0]
    h = 0.5 * h * (1.0 + jax.lax.erf(h * _INV_SQRT2))
    y = jax.lax.dot_general(h.astype(jnp.bfloat16), wproj_ref[0],
                            (((1,), (1,)), ((), ())),
                            preferred_element_type=jnp.float32)
    y = y + bproj_ref[0]
    cv = c_ref[0, 0]
    z = _expmap(ref_ref[sl, :], y, cv)
    lane = jax.lax.broadcasted_iota(jnp.int32, (BT, N_EXPERTS), 1)
    rwb = rw_ref[sl, :]
    w = jnp.sum(jnp.where(lane == e, rwb, 0.0), axis=1, keepdims=True)
    wz = w * z

    @pl.when(e == 0)
    def _():
        out_ref[sl, :] = wz

    @pl.when(e > 0)
    def _():
        out_ref[sl, :] = out_ref[sl, :] + wz


def kernel(x, reference_point, c, Wg, bg, Wfc, bfc, Wproj, bproj):
    b, t, ch = x.shape
    x_flat = x.reshape(b * t, ch)
    ref_flat = reference_point.reshape(b * t, ch)

    wgt = jnp.zeros((N_EMBD, 128), dtype=jnp.float32).at[:, : N_EXPERTS - 1].set(Wg.T)
    bgp = jnp.full((1, 128), -1e30, dtype=jnp.float32).at[0, : N_EXPERTS - 1].set(bg)

    rw = pl.pallas_call(
        _gate_kernel,
        grid=(T // BTG,),
        in_specs=[
            pl.BlockSpec((BTG, N_EMBD), lambda i: (i, 0)),
            pl.BlockSpec((N_EMBD, 128), lambda i: (0, 0)),
            pl.BlockSpec((1, 128), lambda i: (0, 0)),
        ],
        out_specs=pl.BlockSpec((BTG, N_EXPERTS), lambda i: (i, 0)),
        out_shape=jax.ShapeDtypeStruct((T, N_EXPERTS), jnp.float32),
    )(x_flat, wgt, bgp)

    x_bf = x_flat.astype(jnp.bfloat16)
    wfc_bf = Wfc.astype(jnp.bfloat16)
    wproj_bf = Wproj.astype(jnp.bfloat16)
    c2d = c.reshape(1, 1)

    out = pl.pallas_call(
        _dense_kernel,
        grid=(N_EXPERTS, T // BT),
        in_specs=[
            pl.BlockSpec((T, N_EMBD), lambda e, i: (0, 0)),
            pl.BlockSpec((T, N_EMBD), lambda e, i: (0, 0)),
            pl.BlockSpec((T, N_EXPERTS), lambda e, i: (0, 0)),
            pl.BlockSpec((1, 1), lambda e, i: (0, 0)),
            pl.BlockSpec((1, HIDDEN, N_EMBD), lambda e, i: (e, 0, 0)),
            pl.BlockSpec((1, 1, HIDDEN), lambda e, i: (e, 0, 0)),
            pl.BlockSpec((1, N_EMBD, HIDDEN), lambda e, i: (e, 0, 0)),
            pl.BlockSpec((1, 1, N_EMBD), lambda e, i: (e, 0, 0)),
        ],
        out_specs=pl.BlockSpec((T, N_EMBD), lambda e, i: (0, 0)),
        out_shape=jax.ShapeDtypeStruct((T, N_EMBD), jnp.float32),
    )(x_bf, ref_flat, rw, c2d, wfc_bf, bfc.reshape(N_EXPERTS, 1, HIDDEN),
      wproj_bf, bproj.reshape(N_EXPERTS, 1, N_EMBD))

    return (out.reshape(b, t, ch), rw)
